# baseline (speedup 1.0000x reference)
import functools

import jax
import jax.numpy as jnp
from jax import lax
from jax.experimental import pallas as pl
from jax.experimental.pallas import tpu as pltpu
from jax.experimental.pallas import tpu_sc as plsc

BATCH = 16384
HIST = 50
EMBED_DIM = 32
TOTAL = BATCH * HIST

_info = plsc.get_sparse_core_info()
NUM_CORES = _info.num_cores
NUM_SUBCORES = _info.num_subcores
NUM_WORKERS = NUM_CORES * NUM_SUBCORES

CHUNK = 512
CHUNKS_PER_H = BATCH // CHUNK
NUM_CHUNKS = HIST * CHUNKS_PER_H
PER_WORKER = NUM_CHUNKS // NUM_WORKERS
RBUF = 4
WBUF = 2

H_STRIDE = 4 * 128 * 1024
CB_STRIDE = 128 * 1024
PIECE = 4 * 1024


def _transpose_chunk(rows_v, obuf):
    iota = lax.iota(jnp.int32, 16)

    @plsc.parallel_loop(0, EMBED_DIM, unroll=2)
    def _c(c0):
        cdiag = (c0 + iota) & 31
        wbase = cdiag * 128 + (cdiag >> 3) * 3072 + iota
        for g in range(32):
            bt = g >> 3
            bg = g & 7
            vals = plsc.load_gather(rows_v, [bt * 128 + bg * 16 + iota,
                                             cdiag])
            plsc.store_scatter(obuf, [wbase + (bt * 1024 + bg * 16)], vals)


def _run_table(idx_hbm, tab_hbm, out_hbm, ibufs, rbufs, obufs, gsems, wsems,
               wid):
    def chunk_params(i):
        g = wid * PER_WORKER + i
        h = g // CHUNKS_PER_H
        b0 = (g % CHUNKS_PER_H) * CHUNK
        return h, b0

    def idx_load(i, b):
        h, b0 = chunk_params(i)
        pltpu.sync_copy(idx_hbm.at[pl.ds(h * BATCH + b0, CHUNK)], ibufs[b])

    def gather_start(b):
        pltpu.async_copy(tab_hbm.at[ibufs[b]], rbufs[b], gsems[b])

    def gather_wait(b):
        pltpu.make_async_copy(tab_hbm.at[ibufs[b]], rbufs[b], gsems[b]).wait()

    def wb_start(i, b):
        h, b0 = chunk_params(i)
        base = h * H_STRIDE + b0 * 8
        for cb in range(4):
            pltpu.async_copy(obufs[b].at[pl.ds(cb * PIECE, PIECE)],
                             out_hbm.at[pl.ds(base + cb * CB_STRIDE, PIECE)],
                             wsems[b])

    def wb_wait(i, b):
        h, b0 = chunk_params(i)
        base = h * H_STRIDE + b0 * 8
        for cb in range(4):
            pltpu.make_async_copy(
                obufs[b].at[pl.ds(cb * PIECE, PIECE)],
                out_hbm.at[pl.ds(base + cb * CB_STRIDE, PIECE)],
                wsems[b]).wait()

    for b in range(RBUF):
        idx_load(b, b)
        gather_start(b)

    ROUNDS = (PER_WORKER + RBUF - 1) // RBUF

    def body(r, carry):
        for k in range(RBUF):
            j = r * RBUF + k
            ob = k % WBUF

            @pl.when(j < PER_WORKER)
            def _():
                gather_wait(k)

                @pl.when(j + RBUF < PER_WORKER)
                def _():
                    idx_load(j + RBUF, k)

                @pl.when(j >= WBUF)
                def _():
                    wb_wait(j - WBUF, ob)

                _transpose_chunk(rbufs[k], obufs[ob])

                @pl.when(j + RBUF < PER_WORKER)
                def _():
                    gather_start(k)

                wb_start(j, ob)
        return carry

    lax.fori_loop(0, ROUNDS, body, 0)

    for j in range(PER_WORKER - WBUF, PER_WORKER):
        wb_wait(j, j % WBUF)


def _sc_body(widx_hbm, sidx_hbm, wtab_hbm, stab_hbm, wout_hbm, sout_hbm,
             ibufs, rbufs, obufs, gsems, wsems):
    wid = lax.axis_index("s") * NUM_CORES + lax.axis_index("c")
    _run_table(widx_hbm, wtab_hbm, wout_hbm, ibufs, rbufs, obufs, gsems,
               wsems, wid)
    _run_table(sidx_hbm, stab_hbm, sout_hbm, ibufs, rbufs, obufs, gsems,
               wsems, wid)


def kernel(word_indices, synset_indices, word_table, synset_table):
    widx = word_indices.T.reshape(TOTAL).astype(jnp.int32)
    sidx = synset_indices.T.reshape(TOTAL).astype(jnp.int32)

    mesh = plsc.VectorSubcoreMesh(core_axis_name="c", subcore_axis_name="s")
    run = pl.kernel(
        _sc_body,
        mesh=mesh,
        out_type=[
            jax.ShapeDtypeStruct((HIST * EMBED_DIM * BATCH,), jnp.float32),
            jax.ShapeDtypeStruct((HIST * EMBED_DIM * BATCH,), jnp.float32),
        ],
        scratch_types=[
            [pltpu.VMEM((CHUNK,), jnp.int32) for _ in range(RBUF)],
            [pltpu.VMEM((CHUNK, EMBED_DIM), jnp.float32) for _ in range(RBUF)],
            [pltpu.VMEM((4 * PIECE,), jnp.float32) for _ in range(WBUF)],
            [pltpu.SemaphoreType.DMA for _ in range(RBUF)],
            [pltpu.SemaphoreType.DMA for _ in range(WBUF)],
        ],
        compiler_params=pltpu.CompilerParams(use_tc_tiling_on_sc=False,
                                             needs_layout_passes=False,
                                             disable_bounds_checks=True),
    )
    wout_flat, sout_flat = run(widx, sidx, word_table, synset_table)

    def to_logical(flat):
        v = flat.reshape(HIST, 4, 128, 8, 128)
        return v.transpose(2, 4, 0, 1, 3).reshape(BATCH, HIST, EMBED_DIM)

    return (to_logical(wout_flat), to_logical(sout_flat))


# speedup vs baseline: 1.0486x; 1.0486x over previous
import functools

import jax
import jax.numpy as jnp
from jax import lax
from jax.experimental import pallas as pl
from jax.experimental.pallas import tpu as pltpu
from jax.experimental.pallas import tpu_sc as plsc

BATCH = 16384
HIST = 50
EMBED_DIM = 32
TOTAL = BATCH * HIST

_info = plsc.get_sparse_core_info()
NUM_CORES = _info.num_cores
NUM_SUBCORES = _info.num_subcores
NUM_WORKERS = NUM_CORES * NUM_SUBCORES

CHUNK = 1024
CHUNKS_PER_H = BATCH // CHUNK
NUM_CHUNKS = HIST * CHUNKS_PER_H
PER_WORKER = NUM_CHUNKS // NUM_WORKERS
RBUF = 2
NGRP = CHUNK // 128

PIECE = CHUNK * 8


def _transpose_chunk(rows_v, obuf):
    iota = lax.iota(jnp.int32, 16)

    @plsc.parallel_loop(0, EMBED_DIM, unroll=2)
    def _c(c0):
        cdiag = (c0 + iota) & 31
        crow = cdiag >> 3
        ccol = (cdiag & 7) * 128 + iota
        for g in range(2 * NGRP):
            bt = g >> 1
            bg = g & 1
            base = bt * 128 + bg * 64
            dyn = bt * 1024 + bg * 64
            for s in range(4):
                vals = plsc.load_gather(
                    rows_v, [base + s * 16 + iota, cdiag])
                plsc.store_scatter(obuf, [crow, ccol + (dyn + s * 16)], vals)


def _run_table(idx_hbm, tab_hbm, out_hbm, ibufs, rbufs, obuf, gsems, wsem,
               wid):
    def chunk_params(i):
        g = wid * PER_WORKER + i
        h = g // CHUNKS_PER_H
        b0 = (g % CHUNKS_PER_H) * CHUNK
        return h, b0

    def idx_load(i, b):
        h, b0 = chunk_params(i)
        pltpu.sync_copy(idx_hbm.at[pl.ds(h * BATCH + b0, CHUNK)], ibufs[b])

    def gather_start(b):
        pltpu.async_copy(tab_hbm.at[ibufs[b]], rbufs[b], gsems[b])

    def gather_wait(b):
        pltpu.make_async_copy(tab_hbm.at[ibufs[b]], rbufs[b], gsems[b]).wait()

    def wb_dst(i):
        h, b0 = chunk_params(i)
        return out_hbm.at[pl.ds(h * 4, 4), pl.ds(b0 * 8, PIECE)]

    def wb_start(i):
        pltpu.async_copy(obuf, wb_dst(i), wsem)

    def wb_wait(i):
        pltpu.make_async_copy(obuf, wb_dst(i), wsem).wait()

    for b in range(RBUF):
        idx_load(b, b)
        gather_start(b)

    ROUNDS = (PER_WORKER + RBUF - 1) // RBUF

    def body(r, carry):
        for k in range(RBUF):
            j = r * RBUF + k

            @pl.when(j < PER_WORKER)
            def _():
                gather_wait(k)

                @pl.when(j + RBUF < PER_WORKER)
                def _():
                    idx_load(j + RBUF, k)

                @pl.when(j >= 1)
                def _():
                    wb_wait(j - 1)

                _transpose_chunk(rbufs[k], obuf)

                @pl.when(j + RBUF < PER_WORKER)
                def _():
                    gather_start(k)

                wb_start(j)
        return carry

    lax.fori_loop(0, ROUNDS, body, 0)

    wb_wait(PER_WORKER - 1)


def _sc_body(widx_hbm, sidx_hbm, wtab_hbm, stab_hbm, wout_hbm, sout_hbm,
             ibufs, rbufs, obuf, gsems, wsem):
    wid = lax.axis_index("s") * NUM_CORES + lax.axis_index("c")
    _run_table(widx_hbm, wtab_hbm, wout_hbm, ibufs, rbufs, obuf, gsems,
               wsem, wid)
    _run_table(sidx_hbm, stab_hbm, sout_hbm, ibufs, rbufs, obuf, gsems,
               wsem, wid)


def kernel(word_indices, synset_indices, word_table, synset_table):
    widx = word_indices.T.reshape(TOTAL).astype(jnp.int32)
    sidx = synset_indices.T.reshape(TOTAL).astype(jnp.int32)

    mesh = plsc.VectorSubcoreMesh(core_axis_name="c", subcore_axis_name="s")
    run = pl.kernel(
        _sc_body,
        mesh=mesh,
        out_type=[
            jax.ShapeDtypeStruct((HIST * 4, BATCH * 8), jnp.float32),
            jax.ShapeDtypeStruct((HIST * 4, BATCH * 8), jnp.float32),
        ],
        scratch_types=[
            [pltpu.VMEM((CHUNK,), jnp.int32) for _ in range(RBUF)],
            [pltpu.VMEM((CHUNK, EMBED_DIM), jnp.float32) for _ in range(RBUF)],
            pltpu.VMEM((4, PIECE), jnp.float32),
            [pltpu.SemaphoreType.DMA for _ in range(RBUF)],
            pltpu.SemaphoreType.DMA,
        ],
        compiler_params=pltpu.CompilerParams(use_tc_tiling_on_sc=False,
                                             needs_layout_passes=False,
                                             disable_bounds_checks=True),
    )
    wout_flat, sout_flat = run(widx, sidx, word_table, synset_table)

    def to_logical(flat):
        v = flat.reshape(HIST, 4, 128, 8, 128)
        return v.transpose(2, 4, 0, 1, 3).reshape(BATCH, HIST, EMBED_DIM)

    return (to_logical(wout_flat), to_logical(sout_flat))
